# baseline (speedup 1.0000x reference)
import jax
import jax.numpy as jnp
from jax import lax
from jax.experimental import pallas as pl
from jax.experimental.pallas import tpu as pltpu
from jax.experimental.pallas import tpu_sc as plsc

NUM_CORES = 2
NUM_SUBCORES = 16
NUM_WORKERS = NUM_CORES * NUM_SUBCORES
LANES = 16
TC_BLOCK = 1024


def _tc_copy_body(x_ref, m_ref, o_ref, m32_ref):
    o_ref[...] = x_ref[...]
    m32_ref[...] = m_ref[...].astype(jnp.int32)


def _tc_copy(x2, mask):
    s, d = x2.shape
    nblk = s // TC_BLOCK
    return pl.pallas_call(
        _tc_copy_body,
        grid=(nblk,),
        in_specs=[
            pl.BlockSpec((TC_BLOCK, d), lambda i: (i, 0)),
            pl.BlockSpec((1, TC_BLOCK), lambda i: (0, i)),
        ],
        out_specs=[
            pl.BlockSpec((TC_BLOCK, d), lambda i: (i, 0)),
            pl.BlockSpec((1, TC_BLOCK), lambda i: (0, i)),
        ],
        out_shape=[
            jax.ShapeDtypeStruct((s, d), jnp.float32),
            jax.ShapeDtypeStruct((1, s), jnp.int32),
        ],
    )(x2, mask)


def _sc_body(out_hbm, a_hbm, m_hbm, mbuf, midx, grpbuf, sem_m, sem_g, sem_s):
    chunk = a_hbm.shape[0] // NUM_WORKERS
    wid = lax.axis_index("s") * NUM_CORES + lax.axis_index("c")
    base = wid * chunk

    mask_d = pltpu.async_copy(m_hbm.at[pl.ds(base, chunk)], mbuf, sem_m)

    mask_d.wait()
    iota = lax.broadcasted_iota(jnp.int32, (LANES,), 0)
    big = jnp.int32(2**31 - 1)

    cnt = jnp.int32(0)
    minv = jnp.full((LANES,), big, jnp.int32)
    for j in range(chunk // LANES):
        mv = mbuf[pl.ds(j * LANES, LANES)]
        msk = mv != 0
        idxv = iota + (base + j * LANES)
        pos = cnt + jnp.cumsum(jnp.where(msk, 1, 0)) - 1
        plsc.store_scatter(midx, [pos], idxv, mask=msk)
        minv = jnp.minimum(minv, jnp.where(msk, idxv, big))
        cnt = cnt + plsc.all_reduce_population_count(msk)[0]
    min_masked = jnp.min(minv)
    ngroups = (cnt + LANES - 1) // LANES

    def safe_idx(g):
        idxv = midx[pl.ds(g * LANES, LANES)]
        lane = iota + g * LANES
        return jnp.where(lane < cnt, idxv, min_masked)

    @pl.when(ngroups > 0)
    def _():
        def group(g, carry):
            sidx = safe_idx(g)
            pltpu.async_copy(a_hbm.at[sidx], grpbuf, sem_g).wait()
            pltpu.async_copy(grpbuf, out_hbm.at[sidx], sem_s).wait()
            return carry
        lax.fori_loop(0, ngroups, group, jnp.int32(0))


def _sc_overwrite(out_ref, a2, m32):
    s, d = a2.shape
    chunk = s // NUM_WORKERS
    mesh = plsc.VectorSubcoreMesh(
        core_axis_name="c", subcore_axis_name="s",
        num_cores=NUM_CORES, num_subcores=NUM_SUBCORES)
    pl.kernel(
        _sc_body,
        out_type=(),
        mesh=mesh,
        scratch_types=[
            pltpu.VMEM((chunk,), jnp.int32),
            pltpu.VMEM((chunk,), jnp.int32),
            pltpu.VMEM((LANES, d), jnp.float32),
            pltpu.SemaphoreType.DMA,
            pltpu.SemaphoreType.DMA,
            pltpu.SemaphoreType.DMA,
        ],
        compiler_params=pltpu.CompilerParams(needs_layout_passes=False),
    )(out_ref, a2, m32)


@jax.jit
def kernel(x, attack, attack_mask):
    b, s, d = x.shape
    x2 = x.reshape(s, d)
    a2 = attack.astype(x.dtype).reshape(s, d)
    out0, m32 = _tc_copy(x2, attack_mask.reshape(1, s))
    out_ref = jax.new_ref(out0)
    _sc_overwrite(out_ref, a2, m32.reshape(s))
    return out_ref[...].reshape(b, s, d)
